# Initial kernel scaffold; baseline (speedup 1.0000x reference)
import jax
import jax.numpy as jnp
from jax.experimental import pallas as pl


def kernel(boxes, scores):
    raise NotImplementedError("write your pallas kernel here")


# speedup vs baseline: 25.2179x; 25.2179x over previous
import jax
import jax.numpy as jnp
from jax import lax
from jax.experimental import pallas as pl
from jax.experimental.pallas import tpu as pltpu

_N = 20000
_LANES = 128
_ROWS = 160
_PAD = _ROWS * _LANES
_MAX_OUT = 500
_THR = 0.5
_BIG = 2 ** 30


def _nms_body(x1_ref, y1_ref, x2_ref, y2_ref, s_ref, out_ref, supp_ref, area_ref):
    rows = lax.broadcasted_iota(jnp.int32, (_ROWS, _LANES), 0)
    lanes = lax.broadcasted_iota(jnp.int32, (_ROWS, _LANES), 1)
    flat = rows * _LANES + lanes
    supp_ref[...] = (flat >= _N).astype(jnp.float32)
    area_ref[...] = (x2_ref[...] - x1_ref[...]) * (y2_ref[...] - y1_ref[...])
    lane_iota = lax.broadcasted_iota(jnp.int32, (1, _LANES), 1)

    def body(i, _):
        supp = supp_ref[...]
        avail = supp == 0.0
        cand = jnp.min(jnp.where(avail, flat, _BIG))
        valid = cand < _BIG
        idx = jnp.where(valid, cand, 0)
        r = idx // _LANES
        c = idx - r * _LANES
        lmask = lane_iota == c

        def fetch(ref):
            row = ref[pl.ds(r, 1), :]
            return jnp.sum(jnp.where(lmask, row, 0.0), axis=1, keepdims=True)

        bx1 = fetch(x1_ref)
        by1 = fetch(y1_ref)
        bx2 = fetch(x2_ref)
        by2 = fetch(y2_ref)
        bs = fetch(s_ref)

        xx1 = jnp.maximum(bx1, x1_ref[...])
        yy1 = jnp.maximum(by1, y1_ref[...])
        xx2 = jnp.minimum(bx2, x2_ref[...])
        yy2 = jnp.minimum(by2, y2_ref[...])
        w = jnp.maximum(xx2 - xx1, 0.0)
        h = jnp.maximum(yy2 - yy1, 0.0)
        inter = w * h
        area_a = (bx2 - bx1) * (by2 - by1)
        iou = inter / (area_a + area_ref[...] - inter + 1e-6)
        hit = jnp.logical_and(valid, iou >= _THR)
        new_supp = jnp.maximum(supp, hit.astype(jnp.float32))
        new_supp = jnp.maximum(new_supp, (flat == idx).astype(jnp.float32))
        supp_ref[...] = new_supp

        vf = jnp.where(valid, 1.0, 0.0).astype(jnp.float32)
        out_ref[pl.ds(i, 1), 0:1] = bx1 * vf
        out_ref[pl.ds(i, 1), 1:2] = by1 * vf
        out_ref[pl.ds(i, 1), 2:3] = bx2 * vf
        out_ref[pl.ds(i, 1), 3:4] = by2 * vf
        out_ref[pl.ds(i, 1), 4:5] = bs * vf
        return 0

    lax.fori_loop(0, _MAX_OUT, body, 0)


def kernel(boxes, scores):
    order = jnp.argsort(-scores)
    sb = jnp.take(boxes, order, axis=0)
    ss = jnp.take(scores, order, axis=0)
    pad = _PAD - _N

    def plane(v):
        return jnp.pad(v, (0, pad)).reshape(_ROWS, _LANES)

    out = pl.pallas_call(
        _nms_body,
        out_shape=jax.ShapeDtypeStruct((_MAX_OUT, 5), jnp.float32),
        scratch_shapes=[
            pltpu.VMEM((_ROWS, _LANES), jnp.float32),
            pltpu.VMEM((_ROWS, _LANES), jnp.float32),
        ],
    )(plane(sb[:, 0]), plane(sb[:, 1]), plane(sb[:, 2]), plane(sb[:, 3]), plane(ss))
    return out


# speedup vs baseline: 32.5001x; 1.2888x over previous
import jax
import jax.numpy as jnp
from jax import lax
from jax.experimental import pallas as pl
from jax.experimental.pallas import tpu as pltpu

_N = 20000
_LANES = 128
_ROWS = 160
_PAD = _ROWS * _LANES
_MAX_OUT = 500
_KSLOT = 4
_THR = 0.5


def _nms_body(x1_ref, y1_ref, x2_ref, y2_ref, s_ref, out_ref,
              kx1_ref, ky1_ref, kx2_ref, ky2_ref, karea_ref):
    out_ref[...] = jnp.zeros((_MAX_OUT, 5), jnp.float32)
    kx1_ref[...] = jnp.zeros((_KSLOT, _LANES), jnp.float32)
    ky1_ref[...] = jnp.zeros((_KSLOT, _LANES), jnp.float32)
    kx2_ref[...] = jnp.zeros((_KSLOT, _LANES), jnp.float32)
    ky2_ref[...] = jnp.zeros((_KSLOT, _LANES), jnp.float32)
    karea_ref[...] = jnp.zeros((_KSLOT, _LANES), jnp.float32)

    lane_iota = lax.broadcasted_iota(jnp.int32, (1, _LANES), 1)
    slot_rows = lax.broadcasted_iota(jnp.int32, (_KSLOT, _LANES), 0)
    slot_lanes = lax.broadcasted_iota(jnp.int32, (_KSLOT, _LANES), 1)
    slot_iota = slot_rows * _LANES + slot_lanes

    def cond(state):
        p, count = state
        return jnp.logical_and(count < _MAX_OUT, p < _N)

    def body(state):
        p, count = state
        r = p // _LANES
        c = p - r * _LANES
        lmask = lane_iota == c

        def fetch(ref):
            row = ref[pl.ds(r, 1), :]
            return jnp.sum(jnp.where(lmask, row, 0.0), axis=1, keepdims=True)

        bx1 = fetch(x1_ref)
        by1 = fetch(y1_ref)
        bx2 = fetch(x2_ref)
        by2 = fetch(y2_ref)
        bs = fetch(s_ref)

        xx1 = jnp.maximum(kx1_ref[...], bx1)
        yy1 = jnp.maximum(ky1_ref[...], by1)
        xx2 = jnp.minimum(kx2_ref[...], bx2)
        yy2 = jnp.minimum(ky2_ref[...], by2)
        w = jnp.maximum(xx2 - xx1, 0.0)
        h = jnp.maximum(yy2 - yy1, 0.0)
        inter = w * h
        area_a = (bx2 - bx1) * (by2 - by1)
        iou = inter / (area_a + karea_ref[...] - inter + 1e-6)
        hit = jnp.logical_and(iou >= _THR, slot_iota < count)
        keep = jnp.logical_not(jnp.any(hit))

        @pl.when(keep)
        def _():
            onehot = slot_iota == count
            kx1_ref[...] = jnp.where(onehot, bx1, kx1_ref[...])
            ky1_ref[...] = jnp.where(onehot, by1, ky1_ref[...])
            kx2_ref[...] = jnp.where(onehot, bx2, kx2_ref[...])
            ky2_ref[...] = jnp.where(onehot, by2, ky2_ref[...])
            karea_ref[...] = jnp.where(onehot, area_a, karea_ref[...])
            out_ref[pl.ds(count, 1), 0:1] = bx1
            out_ref[pl.ds(count, 1), 1:2] = by1
            out_ref[pl.ds(count, 1), 2:3] = bx2
            out_ref[pl.ds(count, 1), 3:4] = by2
            out_ref[pl.ds(count, 1), 4:5] = bs

        return (p + 1, count + keep.astype(jnp.int32))

    lax.while_loop(cond, body, (jnp.int32(0), jnp.int32(0)))


def kernel(boxes, scores):
    order = jnp.argsort(-scores)
    sb = jnp.take(boxes, order, axis=0)
    ss = jnp.take(scores, order, axis=0)
    pad = _PAD - _N

    def plane(v):
        return jnp.pad(v, (0, pad)).reshape(_ROWS, _LANES)

    out = pl.pallas_call(
        _nms_body,
        out_shape=jax.ShapeDtypeStruct((_MAX_OUT, 5), jnp.float32),
        scratch_shapes=[
            pltpu.VMEM((_KSLOT, _LANES), jnp.float32),
            pltpu.VMEM((_KSLOT, _LANES), jnp.float32),
            pltpu.VMEM((_KSLOT, _LANES), jnp.float32),
            pltpu.VMEM((_KSLOT, _LANES), jnp.float32),
            pltpu.VMEM((_KSLOT, _LANES), jnp.float32),
        ],
    )(plane(sb[:, 0]), plane(sb[:, 1]), plane(sb[:, 2]), plane(sb[:, 3]), plane(ss))
    return out
